# baseline (speedup 1.0000x reference)
import functools

import jax
import jax.numpy as jnp
from jax import lax
from jax.experimental import pallas as pl
from jax.experimental.pallas import tpu as pltpu
from jax.experimental.pallas import tpu_sc as plsc


@functools.lru_cache(maxsize=None)
def _make_pair_gather(V2, D2, B):
    info = plsc.get_sparse_core_info()
    NC, NS, L = info.num_cores, info.num_subcores, info.num_lanes
    NW = NC * NS
    b_per_w = B // NW
    CHUNK = 128
    n_chunks = b_per_w // CHUNK
    mesh = plsc.VectorSubcoreMesh(core_axis_name="c", subcore_axis_name="s")

    @functools.partial(
        pl.kernel,
        mesh=mesh,
        out_type=jax.ShapeDtypeStruct((B, D2), jnp.float32),
        scratch_types=[
            pltpu.VMEM((b_per_w,), jnp.int32),
            pltpu.VMEM((b_per_w,), jnp.int32),
            pltpu.VMEM((CHUNK, D2), jnp.float32),
            pltpu.SemaphoreType.DMA,
        ],
    )
    def gather(table_hbm, idx_hbm, pairs_hbm, idx_v, pidx_v, buf_v, sem):
        wid = lax.axis_index("s") * NC + lax.axis_index("c")
        base = wid * b_per_w
        pltpu.sync_copy(idx_hbm.at[pl.ds(base, b_per_w)], idx_v)
        for c in range(b_per_w // L):
            vec = idx_v[pl.ds(c * L, L)]
            pidx_v[pl.ds(c * L, L)] = vec >> 1

        def chunk_body(j, carry):
            pltpu.async_copy(
                table_hbm.at[pidx_v.at[pl.ds(j * CHUNK, CHUNK)]],
                buf_v,
                sem,
            ).wait()
            pltpu.sync_copy(
                buf_v, pairs_hbm.at[pl.ds(base + j * CHUNK, CHUNK)]
            )
            return carry

        lax.fori_loop(0, n_chunks, chunk_body, 0, unroll=False)

    return gather


@functools.lru_cache(maxsize=None)
def _make_select(B, D):
    BLK = 2048

    def body(parity_ref, pairs_ref, out_ref):
        p = parity_ref[...]
        even = pairs_ref[:, :D]
        odd = pairs_ref[:, D:]
        out_ref[...] = jnp.where(p == 1, odd, even)

    return pl.pallas_call(
        body,
        grid=(B // BLK,),
        in_specs=[
            pl.BlockSpec((BLK, 1), lambda i: (i, 0)),
            pl.BlockSpec((BLK, 2 * D), lambda i: (i, 0)),
        ],
        out_specs=pl.BlockSpec((BLK, D), lambda i: (i, 0)),
        out_shape=jax.ShapeDtypeStruct((B, D), jnp.float32),
    )


def kernel(emb, indices):
    V, D = emb.shape
    (B,) = indices.shape
    idx = indices.astype(jnp.int32)
    emb_pk = jnp.reshape(emb, (V // 2, 2 * D))
    pairs = _make_pair_gather(V // 2, 2 * D, B)(emb_pk, idx)
    parity = (idx & 1).reshape(B, 1)
    return _make_select(B, D)(parity, pairs)


# speedup vs baseline: 1.7575x; 1.7575x over previous
import functools

import jax
import jax.numpy as jnp
from jax import lax
from jax.experimental import pallas as pl
from jax.experimental.pallas import tpu as pltpu
from jax.experimental.pallas import tpu_sc as plsc

NSEM = 4


@functools.lru_cache(maxsize=None)
def _make_gather(V, D, B):
    info = plsc.get_sparse_core_info()
    NC, NS, L = info.num_cores, info.num_subcores, info.num_lanes
    NW = NC * NS
    assert D % L == 0 and B % (8 * NW) == 0
    b_per_w = B // NW
    UNROLL = 16
    n_outer = b_per_w // UNROLL
    mesh = plsc.VectorSubcoreMesh(core_axis_name="c", subcore_axis_name="s")

    @functools.partial(
        pl.kernel,
        mesh=mesh,
        out_type=jax.ShapeDtypeStruct((B, D), jnp.float32),
        scratch_types=[
            pltpu.VMEM((b_per_w,), jnp.int32),
            pltpu.VMEM((b_per_w, D), jnp.float32),
        ]
        + [pltpu.SemaphoreType.DMA] * NSEM,
    )
    def gather(table_hbm, idx_hbm, out_hbm, idx_v, rows_v, *sems):
        wid = lax.axis_index("s") * NC + lax.axis_index("c")
        base = wid * b_per_w
        pltpu.sync_copy(idx_hbm.at[pl.ds(base, b_per_w)], idx_v)

        def fire(i, carry):
            vec = idx_v[pl.ds(i * UNROLL, UNROLL)]
            for k in range(UNROLL):
                t = vec[k]
                pltpu.async_copy(
                    table_hbm.at[pl.ds(t, 1), :],
                    rows_v.at[pl.ds(i * UNROLL + k, 1), :],
                    sems[k % NSEM],
                )
            return carry

        lax.fori_loop(0, n_outer, fire, 0, unroll=False)
        per_sem_rows = b_per_w // NSEM
        for s in range(NSEM):
            pltpu.make_async_copy(
                out_hbm.at[pl.ds(base, per_sem_rows)],
                rows_v.at[pl.ds(s * per_sem_rows, per_sem_rows), :],
                sems[s],
            ).wait()
        pltpu.sync_copy(rows_v, out_hbm.at[pl.ds(base, b_per_w)])

    return gather


def kernel(emb, indices):
    V, D = emb.shape
    (B,) = indices.shape
    return _make_gather(V, D, B)(emb, indices.astype(jnp.int32))
